# baseline (speedup 1.0000x reference)
import functools

import jax
import jax.numpy as jnp
from jax import lax
from jax.experimental import pallas as pl
from jax.experimental.pallas import tpu as pltpu
from jax.experimental.pallas import tpu_sc as plsc

F = 26
V = 100000
E = 16
A = 8
B = 16384
NPAIR = F * (F - 1) // 2

NC, NS = 2, 16
NW = NC * NS
NROWS = B * F
ROWS_PER_W = NROWS // NW
CHUNK = 1664
NCHUNK = ROWS_PER_W // CHUNK

@functools.cache
def _make_sc_gather():
    mesh = plsc.VectorSubcoreMesh(core_axis_name="c", subcore_axis_name="s")

    @functools.partial(
        pl.kernel,
        out_type=jax.ShapeDtypeStruct((NROWS, E), jnp.float32),
        mesh=mesh,
        scratch_types=[
            pltpu.VMEM((CHUNK,), jnp.int32),
            pltpu.VMEM((CHUNK, E), jnp.float32),
            pltpu.SemaphoreType.DMA,
        ],
        compiler_params=pltpu.CompilerParams(use_tc_tiling_on_sc=False),
    )
    def _sc_gather(tbl_hbm, idx_hbm, out_hbm, idx_v, rows_v, sem):
        wid = lax.axis_index("s") * NC + lax.axis_index("c")
        base = wid * ROWS_PER_W

        def step(i, carry):
            off = base + i * CHUNK
            pltpu.sync_copy(idx_hbm.at[pl.ds(off, CHUNK)], idx_v)
            pltpu.async_copy(tbl_hbm.at[idx_v], rows_v, sem).wait()
            pltpu.sync_copy(rows_v, out_hbm.at[pl.ds(off, CHUNK)])
            return carry

        lax.fori_loop(0, NCHUNK, step, 0)

    return _sc_gather


BB = 32


def _afm_body(x_ref, emb_ref, aw_ref, ab_ref, ph_ref, pp_ref, ww_ref, wb_ref,
              out_ref):
    e = emb_ref[...].reshape(BB, F, E)
    parts = [e[:, i + 1:, :] * e[:, i:i + 1, :] for i in range(F - 1)]
    inner = jnp.concatenate(parts, axis=1)
    inner2 = inner.reshape(BB * NPAIR, E)
    att = jnp.maximum(
        jnp.dot(inner2, aw_ref[...], preferred_element_type=jnp.float32)
        + ab_ref[...], 0.0)
    score = jnp.sum(att.reshape(BB, NPAIR, A) * ph_ref[...].reshape(1, 1, A),
                    axis=2)
    m = jnp.max(score, axis=1, keepdims=True)
    ex = jnp.exp(score - m)
    w = ex / jnp.sum(ex, axis=1, keepdims=True)
    att_out = jnp.sum(inner * w[:, :, None], axis=1)
    afm = jnp.sum(att_out * pp_ref[...], axis=1)
    wide = jnp.maximum(
        jnp.sum(x_ref[...] * ww_ref[...], axis=1) + wb_ref[0, 0], 0.0)
    out_ref[...] = jax.nn.sigmoid(wide + afm).reshape(BB, 1)


def _afm_tc(X, emb2, aw, ab_row, ph_row, pp_row, ww, wb2):
    nblk = B // BB
    return pl.pallas_call(
        _afm_body,
        grid=(nblk,),
        in_specs=[
            pl.BlockSpec((BB, F), lambda i: (i, 0)),
            pl.BlockSpec((BB, F * E), lambda i: (i, 0)),
            pl.BlockSpec((E, A), lambda i: (0, 0)),
            pl.BlockSpec((1, A), lambda i: (0, 0)),
            pl.BlockSpec((1, A), lambda i: (0, 0)),
            pl.BlockSpec((1, E), lambda i: (0, 0)),
            pl.BlockSpec((1, F), lambda i: (0, 0)),
            pl.BlockSpec((1, 1), lambda i: (0, 0)),
        ],
        out_specs=pl.BlockSpec((BB, 1), lambda i: (i, 0)),
        out_shape=jax.ShapeDtypeStruct((B, 1), jnp.float32),
    )(X, emb2, aw, ab_row, ph_row, pp_row, ww, wb2)


def kernel(X, tables, attention_W, attention_b, projection_h, projection_p,
           wide_W, wide_b):
    idx = (X.astype(jnp.int32)
           + (jnp.arange(F, dtype=jnp.int32) * V)[None, :]).reshape(-1)
    emb_flat = _make_sc_gather()(tables.reshape(F * V, E), idx)
    emb2 = emb_flat.reshape(B, F * E)
    out2 = _afm_tc(
        X, emb2, attention_W,
        attention_b.reshape(1, A), projection_h.reshape(1, A),
        projection_p.reshape(1, E), wide_W, wide_b.reshape(1, 1))
    return out2.reshape(B)


# speedup vs baseline: 5.7777x; 5.7777x over previous
import functools

import jax
import jax.numpy as jnp
from jax import lax
from jax.experimental import pallas as pl
from jax.experimental.pallas import tpu as pltpu
from jax.experimental.pallas import tpu_sc as plsc

F = 26
V = 100000
E = 16
A = 8
B = 16384
NPAIR = F * (F - 1) // 2

NC, NS = 2, 16
NW = NC * NS
NROWS = B * F
ROWS_PER_W = NROWS // NW
CHUNK = 1664
NCHUNK = ROWS_PER_W // CHUNK

@functools.cache
def _make_sc_gather():
    mesh = plsc.VectorSubcoreMesh(core_axis_name="c", subcore_axis_name="s")

    @functools.partial(
        pl.kernel,
        out_type=jax.ShapeDtypeStruct((NROWS, E), jnp.float32),
        mesh=mesh,
        scratch_types=[
            pltpu.VMEM((CHUNK,), jnp.int32),
            pltpu.VMEM((CHUNK, E), jnp.float32),
            pltpu.SemaphoreType.DMA,
        ],
        compiler_params=pltpu.CompilerParams(use_tc_tiling_on_sc=False),
    )
    def _sc_gather(tbl_hbm, idx_hbm, out_hbm, idx_v, rows_v, sem):
        wid = lax.axis_index("s") * NC + lax.axis_index("c")
        base = wid * ROWS_PER_W

        def step(i, carry):
            off = base + i * CHUNK
            pltpu.sync_copy(idx_hbm.at[pl.ds(off, CHUNK)], idx_v)
            pltpu.async_copy(tbl_hbm.at[idx_v], rows_v, sem).wait()
            pltpu.sync_copy(rows_v, out_hbm.at[pl.ds(off, CHUNK)])
            return carry

        lax.fori_loop(0, NCHUNK, step, 0)

    return _sc_gather


BB = 256
NP13 = 13
PW = 512
IW = NP13 * PW
SW = 416


def _afm_body(x_ref, emb_ref, wt_ref, abt_ref, ht_ref, e32_ref, es_ref,
              mask_ref, pp_ref, ww_ref, wb_ref, out_ref):
    e2 = emb_ref[...]
    zpad = jnp.zeros((BB, PW - F * E), jnp.float32)
    pieces = []
    for p in range(NP13):
        d = (p + 1) * E
        rot = jnp.concatenate([e2[:, d:], e2[:, :d]], axis=1)
        pieces.append(jnp.concatenate([e2 * rot, zpad], axis=1))
    ifull = jnp.concatenate(pieces, axis=1)

    score_gs = []
    for g in range(2 * NP13):
        sl = ifull[:, 256 * g:256 * (g + 1)]
        att_g = jnp.maximum(
            jnp.dot(sl, wt_ref[...], preferred_element_type=jnp.float32)
            + abt_ref[...], 0.0)
        score_gs.append(
            jnp.dot(att_g, ht_ref[...], preferred_element_type=jnp.float32))
    score = jnp.concatenate(score_gs, axis=1) + mask_ref[...]

    m = jnp.max(score, axis=1, keepdims=True)
    ex = jnp.exp(score - m)
    w = ex / jnp.sum(ex, axis=1, keepdims=True)

    att_out = jnp.zeros((BB, E), jnp.float32)
    for p in range(NP13):
        wexp_p = jnp.dot(w[:, 32 * p:32 * p + 32], e32_ref[...],
                         preferred_element_type=jnp.float32)
        u_p = ifull[:, PW * p:PW * (p + 1)] * wexp_p
        att_out = att_out + jnp.dot(u_p, es_ref[...],
                                    preferred_element_type=jnp.float32)
    afm = jnp.sum(att_out * pp_ref[...], axis=1)
    wide = jnp.maximum(
        jnp.sum(x_ref[...] * ww_ref[...], axis=1) + wb_ref[0, 0], 0.0)
    out_ref[...] = jax.nn.sigmoid(wide + afm).reshape(BB, 1)


def _afm_tc(X, emb2, wt, abt, ht, e32, es, mask, pp_row, ww, wb2):
    nblk = B // BB
    full = lambda shp: pl.BlockSpec(shp, lambda i: tuple(0 for _ in shp))
    return pl.pallas_call(
        _afm_body,
        grid=(nblk,),
        in_specs=[
            pl.BlockSpec((BB, F), lambda i: (i, 0)),
            pl.BlockSpec((BB, F * E), lambda i: (i, 0)),
            full((256, 128)),
            full((1, 128)),
            full((128, E)),
            full((32, PW)),
            full((PW, E)),
            full((1, SW)),
            full((1, E)),
            full((1, F)),
            full((1, 1)),
        ],
        out_specs=pl.BlockSpec((BB, 1), lambda i: (i, 0)),
        out_shape=jax.ShapeDtypeStruct((B, 1), jnp.float32),
    )(X, emb2, wt, abt, ht, e32, es, mask, pp_row, ww, wb2)


def kernel(X, tables, attention_W, attention_b, projection_h, projection_p,
           wide_W, wide_b):
    idx = (X.astype(jnp.int32)
           + (jnp.arange(F, dtype=jnp.int32) * V)[None, :]).reshape(-1)
    emb_flat = _make_sc_gather()(tables.reshape(F * V, E), idx)
    emb2 = emb_flat.reshape(B, F * E)

    eye16 = jnp.eye(E, dtype=jnp.float32)
    wt = jnp.kron(eye16, attention_W)
    abt = jnp.tile(attention_b.reshape(1, A), (1, E))
    ht = jnp.kron(eye16, projection_h.reshape(A, 1))
    lane = jnp.arange(PW)
    e32 = ((lane[None, :] // E == jnp.arange(32)[:, None])
           & (lane[None, :] < F * E)).astype(jnp.float32)
    es = (lane[:, None] % E == jnp.arange(E)[None, :]).astype(jnp.float32)
    sl = jnp.arange(SW)
    valid = (sl % 32 < F) & ((sl // 32 < NP13 - 1) | (sl % 32 < NP13))
    mask = jnp.where(valid, 0.0, -1e30).astype(jnp.float32).reshape(1, SW)

    out2 = _afm_tc(X, emb2, wt, abt, ht, e32, es, mask,
                   projection_p.reshape(1, E), wide_W, wide_b.reshape(1, 1))
    return out2.reshape(B)
